# baseline (speedup 1.0000x reference)
import functools

import jax
import jax.numpy as jnp
import numpy as np
from jax import lax
from jax.experimental import pallas as pl
from jax.experimental.pallas import tpu as pltpu
from jax.experimental.pallas import tpu_sc as plsc

_L = 16
_NC = 2
_NS = 16
_NW = _NC * _NS


def _sinusoidal_pe_np(seq_len: int, d_model: int) -> np.ndarray:
    pos = np.arange(seq_len, dtype=np.float32)[:, None]
    i = np.arange(0, d_model, 2, dtype=np.float32)
    div = np.exp(-(np.log(10000.0)) * i / d_model)
    pe = np.zeros((seq_len, d_model), dtype=np.float32)
    pe[:, 0::2] = np.sin(pos * div)
    pe[:, 1::2] = np.cos(pos * div)
    return pe


@functools.partial(jax.jit, static_argnames=("batch", "seq", "d_model"))
def _embed(x_perm, table, pe, *, batch, seq, d_model):
    P = seq // _NW
    G = 8
    C = G * batch
    nchunks = P // G
    NB = 4
    NP = 4

    mesh = plsc.VectorSubcoreMesh(core_axis_name="c", subcore_axis_name="s")

    @functools.partial(
        pl.kernel,
        out_type=jax.ShapeDtypeStruct((batch * seq, d_model), jnp.float32),
        mesh=mesh,
        scratch_types=[
            [pltpu.VMEM((G, d_model), jnp.float32)] * NP,
            [pltpu.VMEM((C, d_model), jnp.float32)] * NB,
            pltpu.VMEM((nchunks, C), jnp.int32),
            [pltpu.SemaphoreType.DMA] * NB,
            [pltpu.SemaphoreType.DMA] * NB,
            [pltpu.SemaphoreType.DMA] * NP,
            pltpu.SemaphoreType.DMA,
        ],
    )
    def body(x_hbm, table_hbm, pe_hbm, out_hbm,
             pes, toks, idx_v, gsems, ssems, pesems, isem):
        wid = lax.axis_index("s") * _NC + lax.axis_index("c")
        pos0 = wid * P
        i_dma = pltpu.async_copy(x_hbm.at[wid], idx_v, isem)

        gathers, pe_dmas, stores = {}, {}, {}

        def issue_pe(c):
            pp = c % NP
            pe_dmas[c] = pltpu.async_copy(
                pe_hbm.at[pl.ds(pos0 + c * G, G)], pes[pp], pesems[pp])

        def issue_gather(c):
            pb = c % NB
            gathers[c] = pltpu.async_copy(
                table_hbm.at[idx_v.at[c]], toks[pb], gsems[pb])
            issue_pe(c)

        issue_pe(0)
        issue_pe(1)
        i_dma.wait()
        gathers[0] = pltpu.async_copy(
            table_hbm.at[idx_v.at[0]], toks[0], gsems[0])
        gathers[1] = pltpu.async_copy(
            table_hbm.at[idx_v.at[1]], toks[1], gsems[1])

        for c in range(nchunks):
            pb = c % NB
            pp = c % NP
            if c + 2 < nchunks:
                if c >= 2:
                    for d in stores[c - 2]:
                        d.wait()
                issue_gather(c + 2)
            gathers[c].wait()
            pe_dmas[c].wait()

            @pl.loop(0, G, unroll=2)
            def _(p):
                for col in range(0, d_model, _L):
                    pe_val = pes[pp][p, pl.ds(col, _L)]
                    for b in range(batch):
                        plsc.addupdate(toks[pb].at[b * G + p, pl.ds(col, _L)],
                                       pe_val)

            stores[c] = [
                pltpu.async_copy(
                    toks[pb].at[pl.ds(b * G, G)],
                    out_hbm.at[pl.ds(
                        pl.multiple_of(b * seq + pos0 + c * G, 8), G)],
                    ssems[pb])
                for b in range(batch)
            ]

        for c in range(max(0, nchunks - 4), nchunks):
            for d in stores[c]:
                d.wait()

    return body(x_perm, table, pe)


def kernel(x, token_table):
    batch, seq = x.shape
    d_model = token_table.shape[1]
    P = seq // _NW
    G = 8
    C = G * batch
    nchunks = P // G

    pe = jnp.asarray(_sinusoidal_pe_np(seq, d_model))
    x_perm = (x.astype(jnp.int32)
               .reshape(batch, _NW, nchunks, G)
               .transpose(1, 2, 0, 3)
               .reshape(_NW, nchunks, C))
    out = _embed(x_perm, token_table, pe,
                 batch=batch, seq=seq, d_model=d_model)
    return out.reshape(batch, seq, d_model)


# speedup vs baseline: 1.1177x; 1.1177x over previous
import functools

import jax
import jax.numpy as jnp
import numpy as np
from jax import lax
from jax.experimental import pallas as pl
from jax.experimental.pallas import tpu as pltpu
from jax.experimental.pallas import tpu_sc as plsc

_L = 16
_NC = 2
_NS = 16
_NW = _NC * _NS


def _sinusoidal_pe_np(seq_len: int, d_model: int) -> np.ndarray:
    pos = np.arange(seq_len, dtype=np.float32)[:, None]
    i = np.arange(0, d_model, 2, dtype=np.float32)
    div = np.exp(-(np.log(10000.0)) * i / d_model)
    pe = np.zeros((seq_len, d_model), dtype=np.float32)
    pe[:, 0::2] = np.sin(pos * div)
    pe[:, 1::2] = np.cos(pos * div)
    return pe


@functools.partial(jax.jit, static_argnames=("batch", "seq", "d_model"))
def _embed(x_perm, table, pe, *, batch, seq, d_model):
    P = seq // _NW
    G = 8
    C = G * batch
    nchunks = P // G
    NB = 4
    NP = 4

    mesh = plsc.VectorSubcoreMesh(core_axis_name="c", subcore_axis_name="s")

    @functools.partial(
        pl.kernel,
        out_type=jax.ShapeDtypeStruct((batch * seq, d_model), jnp.float32),
        mesh=mesh,
        scratch_types=[
            [pltpu.VMEM((G, d_model), jnp.float32)] * NP,
            [pltpu.VMEM((C, d_model), jnp.float32)] * NB,
            pltpu.VMEM((nchunks, C), jnp.int32),
            [pltpu.SemaphoreType.DMA] * NB,
            [pltpu.SemaphoreType.DMA] * NB,
            [pltpu.SemaphoreType.DMA] * NP,
            pltpu.SemaphoreType.DMA,
        ],
    )
    def body(x_hbm, table_hbm, pe_hbm, out_hbm,
             pes, toks, idx_v, gsems, ssems, pesems, isem):
        wid = lax.axis_index("s") * _NC + lax.axis_index("c")
        pos0 = wid * P
        i_dma = pltpu.async_copy(x_hbm.at[wid], idx_v, isem)

        gathers, pe_dmas, stores = {}, {}, {}

        def issue_pe(c):
            pp = c % NP
            pe_dmas[c] = pltpu.async_copy(
                pe_hbm.at[pl.ds(pos0 + c * G, G)], pes[pp], pesems[pp])

        def issue_gather(c):
            pb = c % NB
            gathers[c] = pltpu.async_copy(
                table_hbm.at[idx_v.at[c]], toks[pb], gsems[pb])
            issue_pe(c)

        issue_pe(0)
        issue_pe(1)
        i_dma.wait()
        gathers[0] = pltpu.async_copy(
            table_hbm.at[idx_v.at[0]], toks[0], gsems[0])
        gathers[1] = pltpu.async_copy(
            table_hbm.at[idx_v.at[1]], toks[1], gsems[1])

        for c in range(nchunks):
            pb = c % NB
            pp = c % NP
            if c + 2 < nchunks:
                if c >= 2:
                    for d in stores[c - 2]:
                        d.wait()
                issue_gather(c + 2)
            gathers[c].wait()
            pe_dmas[c].wait()

            @pl.loop(0, G)
            def _(p):
                for col in range(0, d_model, _L):
                    pe_val = pes[pp][p, pl.ds(col, _L)]
                    for b in range(batch):
                        plsc.addupdate(toks[pb].at[b * G + p, pl.ds(col, _L)],
                                       pe_val)

            stores[c] = [
                pltpu.async_copy(
                    toks[pb].at[pl.ds(b * G, G)],
                    out_hbm.at[pl.ds(
                        pl.multiple_of(b * seq + pos0 + c * G, 8), G)],
                    ssems[pb])
                for b in range(batch)
            ]

        for c in range(max(0, nchunks - 4), nchunks):
            for d in stores[c]:
                d.wait()

    return body(x_perm, table, pe)


def kernel(x, token_table):
    batch, seq = x.shape
    d_model = token_table.shape[1]
    P = seq // _NW
    G = 8
    C = G * batch
    nchunks = P // G

    pe = jnp.asarray(_sinusoidal_pe_np(seq, d_model))
    x_perm = (x.astype(jnp.int32)
               .reshape(batch, _NW, nchunks, G)
               .transpose(1, 2, 0, 3)
               .reshape(_NW, nchunks, C))
    out = _embed(x_perm, token_table, pe,
                 batch=batch, seq=seq, d_model=d_model)
    return out.reshape(batch, seq, d_model)
